# Initial kernel scaffold; baseline (speedup 1.0000x reference)
import jax
import jax.numpy as jnp
from jax.experimental import pallas as pl


def kernel(Z, p, alpha, chi):
    raise NotImplementedError("write your pallas kernel here")


# speedup vs baseline: 5.6480x; 5.6480x over previous
import functools

import jax
import jax.numpy as jnp
from jax import lax
from jax.experimental import pallas as pl
from jax.experimental.pallas import tpu as pltpu
from jax.experimental.pallas import tpu_sc as plsc

_MAXZ = 84
_NP = 24
_N = 1048576
_ROW2 = 2 * _NP
_NC, _NS, _L = 2, 16, 16
_NW = _NC * _NS
_PAIRS = _N // 2
_PPW = _PAIRS // _NW
_CHUNK = 512
_NCHUNK = _PPW // _CHUNK
_GSZ = 128
_NG = _CHUNK // _GSZ


def _build_sc_gather():
    mesh = plsc.VectorSubcoreMesh(
        core_axis_name="c", subcore_axis_name="s",
        num_cores=_NC, num_subcores=_NS)

    @functools.partial(
        pl.kernel,
        out_type=jax.ShapeDtypeStruct((_PAIRS, _ROW2), jnp.float32),
        mesh=mesh,
        compiler_params=pltpu.CompilerParams(
            needs_layout_passes=False, use_tc_tiling_on_sc=False),
        scratch_types=[
            pltpu.VMEM((2 * _CHUNK,), jnp.int16),
            pltpu.VMEM((_CHUNK,), jnp.int32),
            pltpu.VMEM((_CHUNK, _ROW2), jnp.float32),
            pltpu.SemaphoreType.DMA,
        ],
    )
    def sc_gather(z_hbm, p2_hbm, out_hbm, z_v, idx_v, rows_v, sem):
        wid = lax.axis_index("s") * _NC + lax.axis_index("c")
        pair0 = wid * _PPW

        def chunk_body(g, carry):
            pbase = pair0 + g * _CHUNK
            pltpu.sync_copy(z_hbm.at[pl.ds(2 * pbase, 2 * _CHUNK)], z_v)
            for k in range(_CHUNK // _L):
                zpair = z_v[pl.ds(2 * _L * k, 2 * _L)]
                ze, zo = plsc.unpack(
                    zpair, format=plsc.PackFormat.INTERLEAVED,
                    preferred_element_type=jnp.int32)
                idx_v[pl.ds(k * _L, _L)] = ze * _MAXZ + zo
            copies = [
                pltpu.async_copy(
                    p2_hbm.at[idx_v.at[pl.ds(j * _GSZ, _GSZ)]],
                    rows_v.at[pl.ds(j * _GSZ, _GSZ)],
                    sem)
                for j in range(_NG)
            ]
            for c in copies:
                c.wait()
            pltpu.sync_copy(rows_v, out_hbm.at[pl.ds(pbase, _CHUNK)])
            return carry

        lax.fori_loop(0, _NCHUNK, chunk_body, 0)

    return sc_gather


_SC_GATHER = _build_sc_gather()


def kernel(Z, p, alpha, chi):
    z16 = Z.astype(jnp.int16)
    p2 = jnp.concatenate(
        [jnp.broadcast_to(p[:, None, :], (_MAXZ, _MAXZ, _NP)),
         jnp.broadcast_to(p[None, :, :], (_MAXZ, _MAXZ, _NP))],
        axis=-1).reshape(_MAXZ * _MAXZ, _ROW2)
    out2 = _SC_GATHER(z16, p2)
    return (out2.reshape(_N, _NP), alpha, chi)


# speedup vs baseline: 10.9191x; 1.9333x over previous
import functools

import jax
import jax.numpy as jnp
from jax import lax
from jax.experimental import pallas as pl
from jax.experimental.pallas import tpu as pltpu
from jax.experimental.pallas import tpu_sc as plsc

_MAXZ = 84
_NP = 24
_N = 1048576
_NC, _NS, _L = 2, 16, 16
_NW = _NC * _NS
_APW = _N // _NW
_CHUNK = 512
_NCHUNK = _APW // _CHUNK
_TBL = _NP * _MAXZ


def _build_sc_gather():
    mesh = plsc.VectorSubcoreMesh(
        core_axis_name="c", subcore_axis_name="s",
        num_cores=_NC, num_subcores=_NS)

    @functools.partial(
        pl.kernel,
        out_type=jax.ShapeDtypeStruct((_NP, _N), jnp.float32),
        mesh=mesh,
        compiler_params=pltpu.CompilerParams(
            needs_layout_passes=False, use_tc_tiling_on_sc=True),
        scratch_types=[
            pltpu.VMEM((_TBL,), jnp.float32),
            pltpu.VMEM((_CHUNK,), jnp.int32),
            pltpu.VMEM((_NP, _CHUNK), jnp.float32),
        ],
    )
    def sc_gather(z_hbm, pt_hbm, out_hbm, pt_v, z_v, blk_v):
        wid = lax.axis_index("s") * _NC + lax.axis_index("c")
        atom0 = wid * _APW
        pltpu.sync_copy(pt_hbm, pt_v)

        def chunk_body(g, carry):
            abase = atom0 + g * _CHUNK
            pltpu.sync_copy(z_hbm.at[pl.ds(abase, _CHUNK)], z_v)
            for a in range(_CHUNK // _L):
                zvec = z_v[pl.ds(a * _L, _L)]
                for j in range(_NP):
                    vals = plsc.load_gather(pt_v, [zvec + (_MAXZ * j)])
                    blk_v[j, pl.ds(a * _L, _L)] = vals
            pltpu.sync_copy(blk_v, out_hbm.at[:, pl.ds(abase, _CHUNK)])
            return carry

        lax.fori_loop(0, _NCHUNK, chunk_body, 0)

    return sc_gather


_SC_GATHER = _build_sc_gather()


def kernel(Z, p, alpha, chi):
    z32 = Z.astype(jnp.int32)
    pt = p.T.reshape(_TBL)
    outT = _SC_GATHER(z32, pt)
    return (outT.T, alpha, chi)


# speedup vs baseline: 28.1732x; 2.5802x over previous
import functools

import jax
import jax.numpy as jnp
from jax import lax
from jax.experimental import pallas as pl
from jax.experimental.pallas import tpu as pltpu
from jax.experimental.pallas import tpu_sc as plsc

_MAXZ = 84
_NP = 24
_N = 1048576
_NC, _NS, _L = 2, 16, 16
_NW = _NC * _NS
_APW = _N // _NW
_CHUNK = 512
_NCHUNK = _APW // _CHUNK
_TBL = _NP * _MAXZ


def _build_sc_gather():
    mesh = plsc.VectorSubcoreMesh(
        core_axis_name="c", subcore_axis_name="s",
        num_cores=_NC, num_subcores=_NS)

    @functools.partial(
        pl.kernel,
        out_type=jax.ShapeDtypeStruct((_NP, _N), jnp.float32),
        mesh=mesh,
        compiler_params=pltpu.CompilerParams(
            needs_layout_passes=False, use_tc_tiling_on_sc=True),
        scratch_types=[
            pltpu.VMEM((_TBL,), jnp.float32),
            pltpu.VMEM((_CHUNK,), jnp.int32),
            pltpu.VMEM((_NP, _CHUNK), jnp.float32),
        ],
    )
    def sc_gather(z_hbm, pt_hbm, out_hbm, pt_v, z_v, blk_v):
        wid = lax.axis_index("s") * _NC + lax.axis_index("c")
        atom0 = wid * _APW
        pltpu.sync_copy(pt_hbm, pt_v)

        def chunk_body(g, carry):
            abase = atom0 + g * _CHUNK
            pltpu.sync_copy(z_hbm.at[pl.ds(abase, _CHUNK)], z_v)

            @plsc.parallel_loop(0, _CHUNK // _L, unroll=4)
            def group_body(a):
                aoff = a * _L
                zvec = z_v[pl.ds(aoff, _L)]
                for j in range(_NP):
                    vals = plsc.load_gather(pt_v, [zvec + (_MAXZ * j)])
                    blk_v[j, pl.ds(aoff, _L)] = vals

            pltpu.sync_copy(blk_v, out_hbm.at[:, pl.ds(abase, _CHUNK)])
            return carry

        lax.fori_loop(0, _NCHUNK, chunk_body, 0)

    return sc_gather


_SC_GATHER = _build_sc_gather()


def kernel(Z, p, alpha, chi):
    z32 = Z.astype(jnp.int32)
    pt = p.T.reshape(_TBL)
    outT = _SC_GATHER(z32, pt)
    return (outT.T, alpha, chi)
